# baseline (speedup 1.0000x reference)
import functools

import jax
import jax.numpy as jnp
from jax import lax
from jax.experimental import pallas as pl
from jax.experimental.pallas import tpu as pltpu
from jax.experimental.pallas import tpu_sc as plsc

NC = 2
NS = 16
NW = NC * NS

CHUNK = 128
GROUP = 13

CC = 1736
UNROLL = 8


def _make_transpose(v, d):
    assert d == 16
    per_w = (v // NW) & ~7
    n_chunk = per_w // CC
    assert n_chunk * CC == per_w
    rem = v - per_w * NW
    assert rem % 8 == 0 and rem <= CC

    mesh = plsc.VectorSubcoreMesh(core_axis_name="c", subcore_axis_name="s")

    @functools.partial(
        pl.kernel,
        mesh=mesh,
        compiler_params=pltpu.CompilerParams(
            use_tc_tiling_on_sc=False, needs_layout_passes=False),
        out_type=jax.ShapeDtypeStruct((v * d,), jnp.float32),
        scratch_types=[
            pltpu.VMEM((2, d, CC), jnp.float32),
            pltpu.VMEM((2, CC * d), jnp.float32),
            pltpu.SemaphoreType.DMA,
            pltpu.SemaphoreType.DMA,
        ],
    )
    def transpose_kernel(tab_soa, out_lin, vin, vout, isem, osem):
        wid = lax.axis_index("s") * NC + lax.axis_index("c")
        base = wid * per_w
        lane = lax.iota(jnp.int32, 16)

        def do_chunk(c0, buf, ncols):
            hs = [pltpu.async_copy(
                    tab_soa.at[r, pl.ds(c0, ncols)],
                    vin.at[buf, r, pl.ds(0, ncols)], isem)
                  for r in range(d)]
            for h in hs:
                h.wait()

            def col_blk(j, carry):
                for u in range(UNROLL):
                    c = j * UNROLL + u
                    col = plsc.load_gather(
                        vin.at[buf], [lane, jnp.zeros((16,), jnp.int32) + c])
                    vout[buf, pl.ds(c * d, d)] = col
                return carry

            lax.fori_loop(0, ncols // UNROLL, col_blk, 0)
            pltpu.async_copy(
                vout.at[buf, pl.ds(0, ncols * d)],
                out_lin.at[pl.ds(c0 * d, ncols * d)], osem).wait()

        def chunk_body(k, carry):
            do_chunk(base + k * CC, 0, CC)
            return carry

        lax.fori_loop(0, n_chunk, chunk_body, 0)

        if rem:
            @pl.when(wid == NW - 1)
            def _():
                do_chunk(v - rem, 1, rem)

    return transpose_kernel


def _make_gather(total, v, d):
    per_w = total // NW
    n_chunk = per_w // CHUNK
    n_group = n_chunk // GROUP
    rows = GROUP * CHUNK
    assert per_w * NW == total and n_chunk * CHUNK == per_w
    assert n_group * GROUP == n_chunk

    mesh = plsc.VectorSubcoreMesh(core_axis_name="c", subcore_axis_name="s")

    @functools.partial(
        pl.kernel,
        mesh=mesh,
        compiler_params=pltpu.CompilerParams(use_tc_tiling_on_sc=False),
        out_type=jax.ShapeDtypeStruct((total, d), jnp.float32),
        scratch_types=[
            pltpu.VMEM((n_chunk, CHUNK), jnp.int32),
            pltpu.VMEM((rows, d), jnp.float32),
            pltpu.SemaphoreType.DMA,
        ],
    )
    def gather_kernel(idx_hbm, tab_hbm, out_hbm, idx_v, rows_v, gsem):
        wid = lax.axis_index("s") * NC + lax.axis_index("c")
        pltpu.sync_copy(idx_hbm.at[pl.ds(wid * n_chunk, n_chunk)], idx_v)

        def group_body(g, carry):
            handles = []
            for b in range(GROUP):
                h = pltpu.async_copy(
                    tab_hbm.at[idx_v.at[g * GROUP + b]],
                    rows_v.at[pl.ds(b * CHUNK, CHUNK)],
                    gsem,
                )
                handles.append(h)
            for h in handles:
                h.wait()
            pltpu.sync_copy(
                rows_v, out_hbm.at[pl.ds(wid * per_w + g * rows, rows)])
            return carry

        lax.fori_loop(0, n_group, group_body, 0)

    return gather_kernel


def kernel(x, table):
    b, f = x.shape
    v, d = table.shape
    total = b * f
    tab_lin = _make_transpose(v, d)(table.T).reshape(v, d)
    idx = x.reshape(total // CHUNK, CHUNK).astype(jnp.int32)
    out = _make_gather(total, v, d)(idx, tab_lin)
    return out.reshape(b, f * d)


# speedup vs baseline: 2.4868x; 2.4868x over previous
import functools

import jax
import jax.numpy as jnp
from jax import lax
from jax.experimental import pallas as pl
from jax.experimental.pallas import tpu as pltpu
from jax.experimental.pallas import tpu_sc as plsc

NC = 2
NS = 16
NW = NC * NS

CHUNK = 128
GROUP = 13


CCV = 1736


def _make_depad(v, d):
    per_w = (v // NW) & ~7
    n_chunk = per_w // CCV
    assert n_chunk * CCV == per_w
    rem = v - per_w * NW
    assert rem % 8 == 0 and rem <= CCV

    mesh = plsc.VectorSubcoreMesh(core_axis_name="c", subcore_axis_name="s")

    @functools.partial(
        pl.kernel,
        mesh=mesh,
        compiler_params=pltpu.CompilerParams(use_tc_tiling_on_sc=False),
        out_type=jax.ShapeDtypeStruct((v, d), jnp.float32),
        scratch_types=[
            pltpu.VMEM((2, CCV, 16), jnp.float32),
            pltpu.SemaphoreType.DMA,
            pltpu.SemaphoreType.DMA,
        ],
    )
    def depad_kernel(tab_pad, out_lin, vstage, isem, osem):
        wid = lax.axis_index("s") * NC + lax.axis_index("c")
        base = wid * per_w

        def do_chunk(v0, buf, n):
            pltpu.async_copy(
                tab_pad.at[pl.ds(v0, n), pl.ds(0, d)],
                vstage.at[buf, pl.ds(0, n)], isem).wait()
            pltpu.async_copy(
                vstage.at[buf, pl.ds(0, n)],
                out_lin.at[pl.ds(v0, n)], osem).wait()

        def chunk_body(k, carry):
            do_chunk(base + k * CCV, 0, CCV)
            return carry

        lax.fori_loop(0, n_chunk, chunk_body, 0)

        if rem:
            @pl.when(wid == NW - 1)
            def _():
                do_chunk(v - rem, 1, rem)

    return depad_kernel


def _make_gather(total, v, d):
    per_w = total // NW
    n_chunk = per_w // CHUNK
    n_group = n_chunk // GROUP
    rows = GROUP * CHUNK
    assert per_w * NW == total and n_chunk * CHUNK == per_w
    assert n_group * GROUP == n_chunk

    mesh = plsc.VectorSubcoreMesh(core_axis_name="c", subcore_axis_name="s")

    @functools.partial(
        pl.kernel,
        mesh=mesh,
        compiler_params=pltpu.CompilerParams(use_tc_tiling_on_sc=False),
        out_type=jax.ShapeDtypeStruct((total, d), jnp.float32),
        scratch_types=[
            pltpu.VMEM((n_chunk, CHUNK), jnp.int32),
            pltpu.VMEM((rows, d), jnp.float32),
            pltpu.SemaphoreType.DMA,
        ],
    )
    def gather_kernel(idx_hbm, tab_hbm, out_hbm, idx_v, rows_v, gsem):
        wid = lax.axis_index("s") * NC + lax.axis_index("c")
        pltpu.sync_copy(idx_hbm.at[pl.ds(wid * n_chunk, n_chunk)], idx_v)

        def group_body(g, carry):
            handles = []
            for b in range(GROUP):
                h = pltpu.async_copy(
                    tab_hbm.at[idx_v.at[g * GROUP + b]],
                    rows_v.at[pl.ds(b * CHUNK, CHUNK)],
                    gsem,
                )
                handles.append(h)
            for h in handles:
                h.wait()
            pltpu.sync_copy(
                rows_v, out_hbm.at[pl.ds(wid * per_w + g * rows, rows)])
            return carry

        lax.fori_loop(0, n_group, group_body, 0)

    return gather_kernel


def kernel(x, table):
    b, f = x.shape
    v, d = table.shape
    total = b * f
    idx = x.reshape(total // CHUNK, CHUNK).astype(jnp.int32)
    tab_pad = jnp.pad(table, ((0, 0), (0, 128 - d)))
    tab_lin = _make_depad(v, d)(tab_pad)
    out = _make_gather(total, v, d)(idx, tab_lin)
    return out.reshape(b, f * d)


# speedup vs baseline: 2.8973x; 1.1651x over previous
import functools

import jax
import jax.numpy as jnp
from jax import lax
from jax.experimental import pallas as pl
from jax.experimental.pallas import tpu as pltpu
from jax.experimental.pallas import tpu_sc as plsc

NC = 2
NS = 16
NW = NC * NS

CHUNK = 128
GROUP = 13


def _make_gather(total, v, d):
    per_w = total // NW
    n_chunk = per_w // CHUNK
    n_group = n_chunk // GROUP
    rows = GROUP * CHUNK
    assert per_w * NW == total and n_chunk * CHUNK == per_w
    assert n_group * GROUP == n_chunk and n_group % 2 == 0

    mesh = plsc.VectorSubcoreMesh(core_axis_name="c", subcore_axis_name="s")

    @functools.partial(
        pl.kernel,
        mesh=mesh,
        compiler_params=pltpu.CompilerParams(use_tc_tiling_on_sc=False),
        out_type=jax.ShapeDtypeStruct((total, d), jnp.float32),
        scratch_types=[
            pltpu.VMEM((n_chunk, CHUNK), jnp.int32),
            pltpu.VMEM((2, rows, d), jnp.float32),
            pltpu.SemaphoreType.DMA,
            pltpu.SemaphoreType.DMA,
        ],
    )
    def gather_kernel(idx_hbm, tab_hbm, out_hbm, idx_v, rows_v, gsem, osem):
        wid = lax.axis_index("s") * NC + lax.axis_index("c")
        pltpu.sync_copy(idx_hbm.at[pl.ds(wid * n_chunk, n_chunk)], idx_v)

        def fire(g, buf):
            return [pltpu.async_copy(
                        tab_hbm.at[idx_v.at[g * GROUP + b]],
                        rows_v.at[buf, pl.ds(b * CHUNK, CHUNK)],
                        gsem)
                    for b in range(GROUP)]

        def write_out(g, buf):
            return pltpu.async_copy(
                rows_v.at[buf],
                out_hbm.at[pl.ds(wid * per_w + g * rows, rows)],
                osem)

        def pair_body(p, carry):
            g0 = p * 2
            h0 = fire(g0, 0)
            for h in h0:
                h.wait()
            w0 = write_out(g0, 0)
            h1 = fire(g0 + 1, 1)
            for h in h1:
                h.wait()
            w1 = write_out(g0 + 1, 1)
            w0.wait()
            w1.wait()
            return carry

        lax.fori_loop(0, n_group // 2, pair_body, 0)

    return gather_kernel


def kernel(x, table):
    b, f = x.shape
    v, d = table.shape
    total = b * f
    idx = x.reshape(total // CHUNK, CHUNK).astype(jnp.int32)
    out = _make_gather(total, v, d)(idx, table)
    return out.reshape(b, f * d)
